# Initial kernel scaffold; baseline (speedup 1.0000x reference)
import jax
import jax.numpy as jnp
from jax.experimental import pallas as pl


def kernel(inputs, emb0, emb1):
    raise NotImplementedError("write your pallas kernel here")


# speedup vs baseline: 6.0475x; 6.0475x over previous
import jax
import jax.numpy as jnp
from jax import lax
from jax.experimental import pallas as pl
from jax.experimental.pallas import tpu as pltpu
from jax.experimental.pallas import tpu_sc as plsc

L = 16
NC, NS = 2, 16
NW = NC * NS

GRIDS = ((65, 257, 257), (33, 129, 129))
ED = 16
NCORNERS = 8
OD = 2 * ED


def _corner_idx_weights(x, y, z, grid):
    T, H, W = grid
    tx = x * float(T - 1)
    ty = y * float(H - 1)
    tz = z * float(W - 1)
    ix = tx.astype(jnp.int32)
    iy = ty.astype(jnp.int32)
    iz = tz.astype(jnp.int32)
    rx = tx - ix.astype(jnp.float32)
    ry = ty - iy.astype(jnp.float32)
    rz = tz - iz.astype(jnp.float32)
    ixp = jnp.minimum(ix + 1, T - 1)
    iyp = jnp.minimum(iy + 1, H - 1)
    izp = jnp.minimum(iz + 1, W - 1)
    u0 = ix * (H * W)
    u1 = ixp * (H * W)
    v0 = iy * W
    v1 = iyp * W
    idxs, wts = [], []
    for u, wx in ((u0, 1.0 - rx), (u1, rx)):
        for v, wy in ((v0, 1.0 - ry), (v1, ry)):
            uv = u + v
            wxy = wx * wy
            for zz, wz in ((iz, 1.0 - rz), (izp, rz)):
                idxs.append(uv + zz)
                wts.append(wxy * wz)
    return idxs, wts


def _make_body(n):
    pt = n // NW
    ngroups = pt // L

    def body(xc, yc, zc, t0, t1, out,
             x_v, y_v, z_v, idx0_v, idx1_v, w_v, rows0_v, rows1_v, outb_v,
             sem):
        wid = lax.axis_index("s") * NC + lax.axis_index("c")
        base = wid * pt
        pltpu.sync_copy(xc.at[pl.ds(base, pt)], x_v)
        pltpu.sync_copy(yc.at[pl.ds(base, pt)], y_v)
        pltpu.sync_copy(zc.at[pl.ds(base, pt)], z_v)

        @pl.loop(0, ngroups)
        def _grp(g):
            o = g * L
            x = x_v[pl.ds(o, L)]
            y = y_v[pl.ds(o, L)]
            z = z_v[pl.ds(o, L)]
            for lvl, (idx_v, tab) in enumerate(
                    ((idx0_v, t0), (idx1_v, t1))):
                idxs, wts = _corner_idx_weights(x, y, z, GRIDS[lvl])
                for c in range(NCORNERS):
                    idx_v[pl.ds(c * L, L)] = idxs[c]
                    w_v[pl.ds(L + (lvl * NCORNERS + c) * L, L)] = wts[c]
            d0 = pltpu.async_copy(t0.at[idx0_v], rows0_v, sem)
            d1 = pltpu.async_copy(t1.at[idx1_v], rows1_v, sem)
            d0.wait()
            d1.wait()
            for p in range(L):
                for lvl, rows_v in enumerate((rows0_v, rows1_v)):
                    acc = None
                    for ci in range(NCORNERS):
                        r = ci * L + p
                        wb = plsc.load_gather(
                            w_v,
                            [jnp.full((L,), L + lvl * NCORNERS * L + r,
                                      jnp.int32)])
                        contrib = wb * rows_v[r, :]
                        acc = contrib if acc is None else acc + contrib
                    outb_v[pl.ds(p * OD + lvl * ED, ED)] = acc
            pltpu.sync_copy(outb_v, out.at[pl.ds((base + o) * OD, L * OD)])

    return body


def kernel(inputs, emb0, emb1):
    n = inputs.shape[0]
    xs = inputs.T
    t0 = emb0.reshape(-1, ED)
    t1 = emb1.reshape(-1, ED)
    pt = n // NW
    mesh = plsc.VectorSubcoreMesh(core_axis_name="c", subcore_axis_name="s")
    run = pl.kernel(
        _make_body(n),
        out_type=jax.ShapeDtypeStruct((n * OD,), jnp.float32),
        mesh=mesh,
        compiler_params=pltpu.CompilerParams(
            needs_layout_passes=False, use_tc_tiling_on_sc=False),
        scratch_types=[
            pltpu.VMEM((pt,), jnp.float32),
            pltpu.VMEM((pt,), jnp.float32),
            pltpu.VMEM((pt,), jnp.float32),
            pltpu.VMEM((NCORNERS * L,), jnp.int32),
            pltpu.VMEM((NCORNERS * L,), jnp.int32),
            pltpu.VMEM((L + 2 * NCORNERS * L,), jnp.float32),
            pltpu.VMEM((NCORNERS * L, ED), jnp.float32),
            pltpu.VMEM((NCORNERS * L, ED), jnp.float32),
            pltpu.VMEM((L * OD,), jnp.float32),
            pltpu.SemaphoreType.DMA,
        ],
    )
    flat = run(xs[0], xs[1], xs[2], t0, t1)
    return flat.reshape(n, OD)


# speedup vs baseline: 6.2481x; 1.0332x over previous
import jax
import jax.numpy as jnp
from jax import lax
from jax.experimental import pallas as pl
from jax.experimental.pallas import tpu as pltpu
from jax.experimental.pallas import tpu_sc as plsc

L = 16
NC, NS = 2, 16
NW = NC * NS

GRIDS = ((65, 257, 257), (33, 129, 129))
ED = 16
NCORNERS = 8
OD = 2 * ED
NR = NCORNERS * L


def _corner_idx_weights(x, y, z, grid):
    T, H, W = grid
    tx = x * float(T - 1)
    ty = y * float(H - 1)
    tz = z * float(W - 1)
    ix = tx.astype(jnp.int32)
    iy = ty.astype(jnp.int32)
    iz = tz.astype(jnp.int32)
    rx = tx - ix.astype(jnp.float32)
    ry = ty - iy.astype(jnp.float32)
    rz = tz - iz.astype(jnp.float32)
    ixp = jnp.minimum(ix + 1, T - 1)
    iyp = jnp.minimum(iy + 1, H - 1)
    izp = jnp.minimum(iz + 1, W - 1)
    u0 = ix * (H * W)
    u1 = ixp * (H * W)
    v0 = iy * W
    v1 = iyp * W
    idxs, wts = [], []
    for u, wx in ((u0, 1.0 - rx), (u1, rx)):
        for v, wy in ((v0, 1.0 - ry), (v1, ry)):
            uv = u + v
            wxy = wx * wy
            for zz, wz in ((iz, 1.0 - rz), (izp, rz)):
                idxs.append(uv + zz)
                wts.append(wxy * wz)
    return idxs, wts


def _make_body(n):
    pt = n // NW
    ngroups = pt // L
    assert ngroups % 2 == 0

    def body(xc, yc, zc, t0, t1, out,
             x_v, y_v, z_v,
             idx0A, idx1A, wA, rows0A, rows1A, outbA,
             idx0B, idx1B, wB, rows0B, rows1B, outbB,
             gsemA, gsemB, osemA, osemB):
        wid = lax.axis_index("s") * NC + lax.axis_index("c")
        base = wid * pt
        pltpu.sync_copy(xc.at[pl.ds(base, pt)], x_v)
        pltpu.sync_copy(yc.at[pl.ds(base, pt)], y_v)
        pltpu.sync_copy(zc.at[pl.ds(base, pt)], z_v)

        def prep(g, idx0, idx1, w):
            o = g * L
            x = x_v[pl.ds(o, L)]
            y = y_v[pl.ds(o, L)]
            z = z_v[pl.ds(o, L)]
            for lvl, idx_v in ((0, idx0), (1, idx1)):
                idxs, wts = _corner_idx_weights(x, y, z, GRIDS[lvl])
                for c in range(NCORNERS):
                    idx_v[pl.ds(c * L, L)] = idxs[c]
                    w[pl.ds(L + (lvl * NCORNERS + c) * L, L)] = wts[c]

        def fire(idx0, idx1, rows0, rows1, sem):
            pltpu.async_copy(t0.at[idx0], rows0, sem)
            pltpu.async_copy(t1.at[idx1], rows1, sem)

        def drain_gather(idx0, idx1, rows0, rows1, sem):
            pltpu.make_async_copy(t0.at[idx0], rows0, sem).wait()
            pltpu.make_async_copy(t1.at[idx1], rows1, sem).wait()

        def drain_out(g, outb, osem):
            @pl.when(g >= 2)
            def _():
                pltpu.make_async_copy(
                    outb, out.at[pl.ds(base * OD, L * OD)], osem).wait()

        def consume(g, w, rows0, rows1, outb, osem):
            drain_out(g, outb, osem)
            for p in range(L):
                for lvl, rows_v in enumerate((rows0, rows1)):
                    acc = None
                    for ci in range(NCORNERS):
                        r = ci * L + p
                        wb = plsc.load_gather(
                            w, [jnp.full((L,), L + lvl * NR + r, jnp.int32)])
                        contrib = wb * rows_v[r, :]
                        acc = contrib if acc is None else acc + contrib
                    outb[pl.ds(p * OD + lvl * ED, ED)] = acc
            pltpu.async_copy(
                outb, out.at[pl.ds((base + g * L) * OD, L * OD)], osem)

        prep(0, idx0A, idx1A, wA)
        fire(idx0A, idx1A, rows0A, rows1A, gsemA)

        @pl.loop(0, ngroups, step=2)
        def _grp(g):
            prep(g + 1, idx0B, idx1B, wB)
            fire(idx0B, idx1B, rows0B, rows1B, gsemB)
            drain_gather(idx0A, idx1A, rows0A, rows1A, gsemA)
            consume(g, wA, rows0A, rows1A, outbA, osemA)

            @pl.when(g + 2 < ngroups)
            def _():
                prep(g + 2, idx0A, idx1A, wA)
                fire(idx0A, idx1A, rows0A, rows1A, gsemA)

            drain_gather(idx0B, idx1B, rows0B, rows1B, gsemB)
            consume(g + 1, wB, rows0B, rows1B, outbB, osemB)

        pltpu.make_async_copy(
            outbA, out.at[pl.ds(base * OD, L * OD)], osemA).wait()
        pltpu.make_async_copy(
            outbB, out.at[pl.ds(base * OD, L * OD)], osemB).wait()

    return body


def kernel(inputs, emb0, emb1):
    n = inputs.shape[0]
    xs = inputs.T
    t0 = emb0.reshape(-1, ED)
    t1 = emb1.reshape(-1, ED)
    pt = n // NW
    mesh = plsc.VectorSubcoreMesh(core_axis_name="c", subcore_axis_name="s")
    buf_set = [
        pltpu.VMEM((NR,), jnp.int32),
        pltpu.VMEM((NR,), jnp.int32),
        pltpu.VMEM((L + 2 * NR,), jnp.float32),
        pltpu.VMEM((NR, ED), jnp.float32),
        pltpu.VMEM((NR, ED), jnp.float32),
        pltpu.VMEM((L * OD,), jnp.float32),
    ]
    run = pl.kernel(
        _make_body(n),
        out_type=jax.ShapeDtypeStruct((n * OD,), jnp.float32),
        mesh=mesh,
        compiler_params=pltpu.CompilerParams(
            needs_layout_passes=False, use_tc_tiling_on_sc=False),
        scratch_types=[
            pltpu.VMEM((pt,), jnp.float32),
            pltpu.VMEM((pt,), jnp.float32),
            pltpu.VMEM((pt,), jnp.float32),
            *buf_set,
            *buf_set,
            pltpu.SemaphoreType.DMA,
            pltpu.SemaphoreType.DMA,
            pltpu.SemaphoreType.DMA,
            pltpu.SemaphoreType.DMA,
        ],
    )
    flat = run(xs[0], xs[1], xs[2], t0, t1)
    return flat.reshape(n, OD)
